# baseline (speedup 1.0000x reference)
import functools

import jax
import jax.numpy as jnp
from jax import lax
from jax.experimental import pallas as pl
from jax.experimental.pallas import tpu as pltpu
from jax.experimental.pallas import tpu_sc as plsc

B = 4096
F = 26
V = 100000
D = 16

NC = 2
NS = 16
OWN = V // NS
SLAB = 6400
FPC = F // NC
SPW = B // NS
NP = 2
PB = B // NP
CAP = 208
UPD = 224
CHUNK = UPD // 2
NDR = 8
ACCROWS = B + NS * NDR

_mesh = plsc.VectorSubcoreMesh(core_axis_name="c", subcore_axis_name="s")


@functools.partial(
    pl.kernel,
    mesh=_mesh,
    out_type=jax.ShapeDtypeStruct((NC, B, 128), jnp.float32),
    compiler_params=pltpu.CompilerParams(
        needs_layout_passes=False, use_tc_tiling_on_sc=True),
    scratch_types=[
        pltpu.VMEM((8, SLAB), jnp.float32),
        pltpu.VMEM((SLAB,), jnp.float32),
        pltpu.VMEM((B,), jnp.int32),
        pltpu.VMEM((UPD,), jnp.int32),
        pltpu.VMEM((UPD,), jnp.int32),
        pltpu.VMEM((UPD,), jnp.int32),
        pltpu.VMEM((UPD,), jnp.int32),
        pltpu.VMEM((UPD, 128), jnp.float32),
        pltpu.VMEM_SHARED((ACCROWS, 128), jnp.float32),
        pltpu.SemaphoreType.DMA,
        pltpu.SemaphoreType.DMA,
    ],
)
def _fm_scan(xt_hbm, v4_hbm, wt_hbm, part_hbm,
             slab, wslab, xidx, sidx0, sidx1, vloc0, vloc1, upd, acc,
             sem0, sem1):
    cid = lax.axis_index("c")
    sid = lax.axis_index("s")
    own0 = sid * OWN
    stage0 = pl.multiple_of(own0 // 128 * 128, 128)
    dump = B + sid * NDR
    lane = lax.iota(jnp.int32, 16)
    zero16 = jnp.zeros((16,), jnp.float32)

    def zinit(i, carry):
        upd[i, pl.ds(0, 16)] = zero16
        upd[i, pl.ds(16, 16)] = zero16
        upd[i, pl.ds(32, 16)] = zero16
        upd[i, pl.ds(48, 16)] = zero16
        upd[i, pl.ds(64, 16)] = zero16
        upd[i, pl.ds(80, 16)] = zero16
        upd[i, pl.ds(96, 16)] = zero16
        upd[i, pl.ds(112, 16)] = zero16
        return carry
    lax.fori_loop(0, UPD, zinit, 0)

    def zinit2(i, carry):
        vloc0[pl.ds(i * 16, 16)] = jnp.zeros((16,), jnp.int32)
        vloc1[pl.ds(i * 16, 16)] = jnp.zeros((16,), jnp.int32)
        return carry
    lax.fori_loop(0, UPD // 16, zinit2, 0)

    pltpu.sync_copy(upd.at[pl.ds(0, UPD), :], acc.at[pl.ds(sid * SPW, UPD)])
    pltpu.sync_copy(upd.at[pl.ds(0, SPW - UPD), :],
                    acc.at[pl.ds(sid * SPW + UPD, SPW - UPD)])
    pltpu.sync_copy(upd.at[pl.ds(0, NDR), :], acc.at[pl.ds(dump, NDR)])
    plsc.subcore_barrier()

    def per_field(fi, carry):
        f = cid * FPC + fi
        pltpu.sync_copy(xt_hbm.at[f], xidx)
        pltpu.sync_copy(wt_hbm.at[f, pl.ds(stage0, SLAB)], wslab)

        dpat = lane % 8
        hpat = lane // 8

        sidxs = [sidx0, sidx1]
        vlocs = [vloc0, vloc1]
        cnts = []
        for p in range(NP):
            sidx = sidxs[p]
            vloc = vlocs[p]

            def sfill(i, carry):
                sidx[pl.ds(i * 16, 16)] = dump + lane % NDR
                return carry
            lax.fori_loop(0, UPD // 16, sfill, 0)

            def bucket(c, cnt):
                v = xidx[pl.ds(p * PB + c * 16, 16)]
                rel = v - own0
                msk = (rel >= 0) & (rel < OWN)
                sids = p * PB + c * 16 + lane
                plsc.store_compressed(sidx.at[pl.ds(cnt, 16)], sids,
                                      mask=msk)
                plsc.store_compressed(vloc.at[pl.ds(cnt, 16)], v - stage0,
                                      mask=msk)
                pc = plsc.all_reduce_population_count(msk)
                return jnp.minimum(cnt + pc[0], CAP)
            cnts.append(lax.fori_loop(0, PB // 16, bucket, jnp.int32(0)))

        zf16 = jnp.zeros((16,), jnp.float32)
        for h in range(2):
            pltpu.sync_copy(v4_hbm.at[f, h, :, pl.ds(stage0, SLAB)], slab)
            for p in range(NP):
                vloc = vlocs[p]

                def build(it, carry):
                    rowpat = it * 2 + hpat
                    vpair = plsc.load_gather(vloc, [rowpat])
                    vals = plsc.load_gather(slab, [dpat, vpair])
                    plsc.store_scatter(upd, [rowpat, h * 8 + dpat], vals)
                    plsc.store_scatter(upd, [rowpat, h * 8 + 16 + dpat],
                                       vals * vals)
                    oh = 8 - h * 8
                    plsc.store_scatter(upd, [rowpat, oh + dpat], zf16)
                    plsc.store_scatter(upd, [rowpat, oh + 16 + dpat], zf16)
                    if h == 0:
                        wvals = plsc.load_gather(wslab, [vpair])
                        plsc.store_scatter(upd, [rowpat, dpat + 32], wvals,
                                           mask=dpat == 0)
                    else:
                        plsc.store_scatter(upd, [rowpat, dpat + 32], zf16,
                                           mask=dpat == 0)
                    return carry
                lax.fori_loop(0, UPD // 2, build, 0)

                @pl.when(cnts[p] > 0)
                def _scat():
                    pltpu.sync_copy(upd, acc.at[sidxs[p]], add=True)
        return carry

    lax.fori_loop(0, FPC, per_field, 0)
    plsc.subcore_barrier()
    pltpu.sync_copy(acc.at[pl.ds(sid * SPW, SPW)],
                    part_hbm.at[cid, pl.ds(sid * SPW, SPW)])


@functools.partial(
    pl.kernel,
    mesh=_mesh,
    out_type=jax.ShapeDtypeStruct((B,), jnp.float32),
    compiler_params=pltpu.CompilerParams(
        needs_layout_passes=False, use_tc_tiling_on_sc=True),
    scratch_types=[
        pltpu.VMEM((B // 32, 128), jnp.float32),
        pltpu.VMEM((B // 32, 128), jnp.float32),
        pltpu.VMEM((B // 32,), jnp.float32),
        pltpu.SemaphoreType.DMA,
    ],
)
def _fm_combine(part_hbm, out_hbm, rows0, rows1, outv, sem):
    wid = lax.axis_index("s") * NC + lax.axis_index("c")
    n = B // 32
    s0 = wid * n
    lane = lax.iota(jnp.int32, 16)
    lane0 = lane == 0

    pltpu.sync_copy(part_hbm.at[0, pl.ds(s0, n)], rows0)
    pltpu.sync_copy(part_hbm.at[1, pl.ds(s0, n)], rows1)

    def body(i, carry):
        a = rows0[i, pl.ds(0, 16)] + rows1[i, pl.ds(0, 16)]
        q = rows0[i, pl.ds(16, 16)] + rows1[i, pl.ds(16, 16)]
        wv = rows0[i, pl.ds(32, 16)] + rows1[i, pl.ds(32, 16)]
        score = 0.5 * jnp.sum(a * a - q) + jnp.sum(wv)
        plsc.store_scatter(outv, [jnp.full((16,), i, jnp.int32)],
                           jnp.full((16,), score, jnp.float32), mask=lane0)
        return carry

    lax.fori_loop(0, n, body, 0)
    pltpu.sync_copy(outv, out_hbm.at[pl.ds(s0, n)])


def kernel(X, table_v, table_w, bias):
    xt = X.T.astype(jnp.int32)
    v4 = jnp.transpose(table_v, (0, 2, 1)).reshape(F, 2, 8, V)
    wt = table_w.reshape(F, V)
    part = _fm_scan(xt, v4, wt)
    score = _fm_combine(part)
    return score + bias[0]


# speedup vs baseline: 1.2976x; 1.2976x over previous
import functools

import jax
import jax.numpy as jnp
from jax import lax
from jax.experimental import pallas as pl
from jax.experimental.pallas import tpu as pltpu
from jax.experimental.pallas import tpu_sc as plsc

B = 4096
F = 26
V = 100000
D = 16

NC = 2
NS = 16
OWN = V // NS
SLAB = 6400
FPC = F // NC
SPW = B // NS
NP = 2
PB = B // NP
CAP = 208
UPD = 224
CHUNK = UPD // 2
NDR = 8
ACCROWS = B + NS * NDR

_mesh = plsc.VectorSubcoreMesh(core_axis_name="c", subcore_axis_name="s")


@functools.partial(
    pl.kernel,
    mesh=_mesh,
    out_type=jax.ShapeDtypeStruct((NC, B, 128), jnp.float32),
    compiler_params=pltpu.CompilerParams(
        needs_layout_passes=False, use_tc_tiling_on_sc=True),
    scratch_types=[
        pltpu.VMEM((8, SLAB), jnp.float32),
        pltpu.VMEM((SLAB,), jnp.float32),
        pltpu.VMEM((B,), jnp.int32),
        pltpu.VMEM((UPD,), jnp.int32),
        pltpu.VMEM((UPD,), jnp.int32),
        pltpu.VMEM((UPD,), jnp.int32),
        pltpu.VMEM((UPD,), jnp.int32),
        pltpu.VMEM((UPD, 128), jnp.float32),
        pltpu.VMEM_SHARED((ACCROWS, 128), jnp.float32),
        pltpu.SemaphoreType.DMA,
        pltpu.SemaphoreType.DMA,
    ],
)
def _fm_scan(xt_hbm, v4_hbm, wt_hbm, part_hbm,
             slab, wslab, xidx, sidx0, sidx1, vloc0, vloc1, upd, acc,
             sem0, sem1):
    cid = lax.axis_index("c")
    sid = lax.axis_index("s")
    own0 = sid * OWN
    stage0 = pl.multiple_of(own0 // 128 * 128, 128)
    dump = B + sid * NDR
    lane = lax.iota(jnp.int32, 16)
    zero16 = jnp.zeros((16,), jnp.float32)

    def zinit(i, carry):
        upd[i, pl.ds(0, 16)] = zero16
        upd[i, pl.ds(16, 16)] = zero16
        upd[i, pl.ds(32, 16)] = zero16
        upd[i, pl.ds(48, 16)] = zero16
        upd[i, pl.ds(64, 16)] = zero16
        upd[i, pl.ds(80, 16)] = zero16
        upd[i, pl.ds(96, 16)] = zero16
        upd[i, pl.ds(112, 16)] = zero16
        return carry
    lax.fori_loop(0, UPD, zinit, 0)

    def zinit2(i, carry):
        vloc0[pl.ds(i * 16, 16)] = jnp.zeros((16,), jnp.int32)
        vloc1[pl.ds(i * 16, 16)] = jnp.zeros((16,), jnp.int32)
        return carry
    lax.fori_loop(0, UPD // 16, zinit2, 0)

    pltpu.sync_copy(upd.at[pl.ds(0, UPD), :], acc.at[pl.ds(sid * SPW, UPD)])
    pltpu.sync_copy(upd.at[pl.ds(0, SPW - UPD), :],
                    acc.at[pl.ds(sid * SPW + UPD, SPW - UPD)])
    pltpu.sync_copy(upd.at[pl.ds(0, NDR), :], acc.at[pl.ds(dump, NDR)])
    plsc.subcore_barrier()

    def per_field(fi, carry):
        f = cid * FPC + fi
        cp_slab = pltpu.async_copy(
            v4_hbm.at[f, 0, :, pl.ds(stage0, SLAB)], slab, sem1)
        pltpu.sync_copy(xt_hbm.at[f], xidx)
        pltpu.sync_copy(wt_hbm.at[f, pl.ds(stage0, SLAB)], wslab)

        dpat = lane % 8
        hpat = lane // 8

        sidxs = [sidx0, sidx1]
        vlocs = [vloc0, vloc1]
        cnts = []
        for p in range(NP):
            sidx = sidxs[p]
            vloc = vlocs[p]

            def sfill(i, carry):
                sidx[pl.ds(i * 16, 16)] = dump + lane % NDR
                return carry
            lax.fori_loop(0, UPD // 16, sfill, 0)

            def bucket(c, cnt):
                v = xidx[pl.ds(p * PB + c * 16, 16)]
                rel = v - own0
                msk = (rel >= 0) & (rel < OWN)
                sids = p * PB + c * 16 + lane
                plsc.store_compressed(sidx.at[pl.ds(cnt, 16)], sids,
                                      mask=msk)
                plsc.store_compressed(vloc.at[pl.ds(cnt, 16)], v - stage0,
                                      mask=msk)
                pc = plsc.all_reduce_population_count(msk)
                return jnp.minimum(cnt + pc[0], CAP)
            cnts.append(lax.fori_loop(0, PB // 16, bucket, jnp.int32(0)))

        zf16 = jnp.zeros((16,), jnp.float32)
        for h in range(2):
            if h == 0:
                cp_slab.wait()
            else:
                pltpu.sync_copy(v4_hbm.at[f, h, :, pl.ds(stage0, SLAB)], slab)
            for p in range(NP):
                vloc = vlocs[p]

                def build(it, carry):
                    rowpat = it * 2 + hpat
                    vpair = plsc.load_gather(vloc, [rowpat])
                    vals = plsc.load_gather(slab, [dpat, vpair])
                    plsc.store_scatter(upd, [rowpat, h * 8 + dpat], vals)
                    plsc.store_scatter(upd, [rowpat, h * 8 + 16 + dpat],
                                       vals * vals)
                    oh = 8 - h * 8
                    plsc.store_scatter(upd, [rowpat, oh + dpat], zf16)
                    plsc.store_scatter(upd, [rowpat, oh + 16 + dpat], zf16)
                    if h == 0:
                        wvals = plsc.load_gather(wslab, [vpair])
                        plsc.store_scatter(upd, [rowpat, dpat + 32], wvals,
                                           mask=dpat == 0)
                    else:
                        plsc.store_scatter(upd, [rowpat, dpat + 32], zf16,
                                           mask=dpat == 0)
                    return carry
                lax.fori_loop(0, (cnts[p] + 1) // 2, build, 0)

                for j in range(UPD // 32):
                    @pl.when(cnts[p] > j * 32)
                    def _scat():
                        pltpu.sync_copy(
                            upd.at[pl.ds(j * 32, 32), :],
                            acc.at[sidxs[p].at[pl.ds(j * 32, 32)]], add=True)
        return carry

    lax.fori_loop(0, FPC, per_field, 0)
    plsc.subcore_barrier()
    pltpu.sync_copy(acc.at[pl.ds(sid * SPW, SPW)],
                    part_hbm.at[cid, pl.ds(sid * SPW, SPW)])


@functools.partial(
    pl.kernel,
    mesh=_mesh,
    out_type=jax.ShapeDtypeStruct((B,), jnp.float32),
    compiler_params=pltpu.CompilerParams(
        needs_layout_passes=False, use_tc_tiling_on_sc=True),
    scratch_types=[
        pltpu.VMEM((B // 32, 128), jnp.float32),
        pltpu.VMEM((B // 32, 128), jnp.float32),
        pltpu.VMEM((B // 32,), jnp.float32),
        pltpu.SemaphoreType.DMA,
    ],
)
def _fm_combine(part_hbm, out_hbm, rows0, rows1, outv, sem):
    wid = lax.axis_index("s") * NC + lax.axis_index("c")
    n = B // 32
    s0 = wid * n
    lane = lax.iota(jnp.int32, 16)
    lane0 = lane == 0

    pltpu.sync_copy(part_hbm.at[0, pl.ds(s0, n)], rows0)
    pltpu.sync_copy(part_hbm.at[1, pl.ds(s0, n)], rows1)

    def body(i, carry):
        a = rows0[i, pl.ds(0, 16)] + rows1[i, pl.ds(0, 16)]
        q = rows0[i, pl.ds(16, 16)] + rows1[i, pl.ds(16, 16)]
        wv = rows0[i, pl.ds(32, 16)] + rows1[i, pl.ds(32, 16)]
        score = 0.5 * jnp.sum(a * a - q) + jnp.sum(wv)
        plsc.store_scatter(outv, [jnp.full((16,), i, jnp.int32)],
                           jnp.full((16,), score, jnp.float32), mask=lane0)
        return carry

    lax.fori_loop(0, n, body, 0)
    pltpu.sync_copy(outv, out_hbm.at[pl.ds(s0, n)])


def kernel(X, table_v, table_w, bias):
    xt = X.T.astype(jnp.int32)
    v4 = jnp.transpose(table_v, (0, 2, 1)).reshape(F, 2, 8, V)
    wt = table_w.reshape(F, V)
    part = _fm_scan(xt, v4, wt)
    score = _fm_combine(part)
    return score + bias[0]


# speedup vs baseline: 1.3582x; 1.0467x over previous
import functools

import jax
import jax.numpy as jnp
from jax import lax
from jax.experimental import pallas as pl
from jax.experimental.pallas import tpu as pltpu
from jax.experimental.pallas import tpu_sc as plsc

B = 4096
F = 26
V = 100000
D = 16

NC = 2
NS = 16
OWN = V // NS
SLAB = 6400
FPC = F // NC
SPW = B // NS
NP = 2
PB = B // NP
CAP = 208
UPD = 224
CHUNK = UPD // 2
NDR = 8
ACCROWS = B + NS * NDR

_mesh = plsc.VectorSubcoreMesh(core_axis_name="c", subcore_axis_name="s")


@functools.partial(
    pl.kernel,
    mesh=_mesh,
    out_type=jax.ShapeDtypeStruct((NC, B, 128), jnp.float32),
    compiler_params=pltpu.CompilerParams(
        needs_layout_passes=False, use_tc_tiling_on_sc=True),
    scratch_types=[
        pltpu.VMEM((8, SLAB), jnp.float32),
        pltpu.VMEM((SLAB,), jnp.float32),
        pltpu.VMEM((B,), jnp.int32),
        pltpu.VMEM((UPD,), jnp.int32),
        pltpu.VMEM((UPD,), jnp.int32),
        pltpu.VMEM((UPD,), jnp.int32),
        pltpu.VMEM((UPD,), jnp.int32),
        pltpu.VMEM((UPD, 128), jnp.float32),
        pltpu.VMEM_SHARED((ACCROWS, 128), jnp.float32),
        pltpu.SemaphoreType.DMA,
        pltpu.SemaphoreType.DMA,
    ],
)
def _fm_scan(xt_hbm, v4_hbm, wt_hbm, part_hbm,
             slab, wslab, xidx, sidx0, sidx1, vloc0, vloc1, upd, acc,
             sem0, sem1):
    cid = lax.axis_index("c")
    sid = lax.axis_index("s")
    own0 = sid * OWN
    stage0 = pl.multiple_of(own0 // 128 * 128, 128)
    dump = B + sid * NDR
    lane = lax.iota(jnp.int32, 16)
    zero16 = jnp.zeros((16,), jnp.float32)

    def zinit(i, carry):
        upd[i, pl.ds(0, 16)] = zero16
        upd[i, pl.ds(16, 16)] = zero16
        upd[i, pl.ds(32, 16)] = zero16
        upd[i, pl.ds(48, 16)] = zero16
        upd[i, pl.ds(64, 16)] = zero16
        upd[i, pl.ds(80, 16)] = zero16
        upd[i, pl.ds(96, 16)] = zero16
        upd[i, pl.ds(112, 16)] = zero16
        return carry
    lax.fori_loop(0, UPD, zinit, 0)

    def zinit2(i, carry):
        vloc0[pl.ds(i * 16, 16)] = jnp.zeros((16,), jnp.int32)
        vloc1[pl.ds(i * 16, 16)] = jnp.zeros((16,), jnp.int32)
        return carry
    lax.fori_loop(0, UPD // 16, zinit2, 0)

    pltpu.sync_copy(upd.at[pl.ds(0, UPD), :], acc.at[pl.ds(sid * SPW, UPD)])
    pltpu.sync_copy(upd.at[pl.ds(0, SPW - UPD), :],
                    acc.at[pl.ds(sid * SPW + UPD, SPW - UPD)])
    pltpu.sync_copy(upd.at[pl.ds(0, NDR), :], acc.at[pl.ds(dump, NDR)])
    plsc.subcore_barrier()

    def per_field(fi, carry):
        f = cid * FPC + fi
        cp_slab = pltpu.async_copy(
            v4_hbm.at[f, 0, :, pl.ds(stage0, SLAB)], slab, sem1)
        pltpu.sync_copy(xt_hbm.at[f], xidx)
        pltpu.sync_copy(wt_hbm.at[f, pl.ds(stage0, SLAB)], wslab)

        dpat = lane % 8
        hpat = lane // 8

        sidxs = [sidx0, sidx1]
        vlocs = [vloc0, vloc1]
        cnts = []
        for p in range(NP):
            sidx = sidxs[p]
            vloc = vlocs[p]

            def sfill(i, carry):
                sidx[pl.ds(i * 16, 16)] = dump + lane % NDR
                return carry
            lax.fori_loop(0, UPD // 16, sfill, 0)

            def bucket(c, cnt):
                v = xidx[pl.ds(p * PB + c * 16, 16)]
                rel = v - own0
                msk = (rel >= 0) & (rel < OWN)
                sids = p * PB + c * 16 + lane
                plsc.store_compressed(sidx.at[pl.ds(cnt, 16)], sids,
                                      mask=msk)
                plsc.store_compressed(vloc.at[pl.ds(cnt, 16)], v - stage0,
                                      mask=msk)
                pc = plsc.all_reduce_population_count(msk)
                return jnp.minimum(cnt + pc[0], CAP)
            cnts.append(lax.fori_loop(0, PB // 16, bucket, jnp.int32(0)))

        zf16 = jnp.zeros((16,), jnp.float32)
        for h in range(2):
            if h == 0:
                cp_slab.wait()
            else:
                pltpu.sync_copy(v4_hbm.at[f, h, :, pl.ds(stage0, SLAB)], slab)
            for p in range(NP):
                vloc = vlocs[p]

                def build(it, carry):
                    rowpat = it * 2 + hpat
                    vpair = plsc.load_gather(vloc, [rowpat])
                    vals = plsc.load_gather(slab, [dpat, vpair])
                    plsc.store_scatter(upd, [rowpat, h * 8 + dpat], vals)
                    plsc.store_scatter(upd, [rowpat, h * 8 + 16 + dpat],
                                       vals * vals)
                    oh = 8 - h * 8
                    plsc.store_scatter(upd, [rowpat, oh + dpat], zf16)
                    plsc.store_scatter(upd, [rowpat, oh + 16 + dpat], zf16)
                    if h == 0:
                        wvals = plsc.load_gather(wslab, [vpair])
                        plsc.store_scatter(upd, [rowpat, dpat + 32], wvals,
                                           mask=dpat == 0)
                    else:
                        plsc.store_scatter(upd, [rowpat, dpat + 32], zf16,
                                           mask=dpat == 0)
                    return carry
                lax.fori_loop(0, (cnts[p] + 1) // 2, build, 0)

                for j in range(UPD // 32):
                    @pl.when(cnts[p] > j * 32)
                    def _scat():
                        pltpu.make_async_copy(
                            upd.at[pl.ds(j * 32, 32), :],
                            acc.at[sidxs[p].at[pl.ds(j * 32, 32)]],
                            sem0).start(add=True)
                for j in range(UPD // 32):
                    @pl.when(cnts[p] > j * 32)
                    def _drain():
                        pltpu.make_async_copy(
                            upd.at[pl.ds(j * 32, 32), :],
                            acc.at[sidxs[p].at[pl.ds(j * 32, 32)]],
                            sem0).wait()
        return carry

    lax.fori_loop(0, FPC, per_field, 0)
    plsc.subcore_barrier()
    pltpu.sync_copy(acc.at[pl.ds(sid * SPW, SPW)],
                    part_hbm.at[cid, pl.ds(sid * SPW, SPW)])


@functools.partial(
    pl.kernel,
    mesh=_mesh,
    out_type=jax.ShapeDtypeStruct((B,), jnp.float32),
    compiler_params=pltpu.CompilerParams(
        needs_layout_passes=False, use_tc_tiling_on_sc=True),
    scratch_types=[
        pltpu.VMEM((B // 32, 128), jnp.float32),
        pltpu.VMEM((B // 32, 128), jnp.float32),
        pltpu.VMEM((B // 32,), jnp.float32),
        pltpu.SemaphoreType.DMA,
    ],
)
def _fm_combine(part_hbm, out_hbm, rows0, rows1, outv, sem):
    wid = lax.axis_index("s") * NC + lax.axis_index("c")
    n = B // 32
    s0 = wid * n
    lane = lax.iota(jnp.int32, 16)
    lane0 = lane == 0

    pltpu.sync_copy(part_hbm.at[0, pl.ds(s0, n)], rows0)
    pltpu.sync_copy(part_hbm.at[1, pl.ds(s0, n)], rows1)

    def body(i, carry):
        a = rows0[i, pl.ds(0, 16)] + rows1[i, pl.ds(0, 16)]
        q = rows0[i, pl.ds(16, 16)] + rows1[i, pl.ds(16, 16)]
        wv = rows0[i, pl.ds(32, 16)] + rows1[i, pl.ds(32, 16)]
        score = 0.5 * jnp.sum(a * a - q) + jnp.sum(wv)
        plsc.store_scatter(outv, [jnp.full((16,), i, jnp.int32)],
                           jnp.full((16,), score, jnp.float32), mask=lane0)
        return carry

    lax.fori_loop(0, n, body, 0)
    pltpu.sync_copy(outv, out_hbm.at[pl.ds(s0, n)])


def kernel(X, table_v, table_w, bias):
    xt = X.T.astype(jnp.int32)
    v4 = jnp.transpose(table_v, (0, 2, 1)).reshape(F, 2, 8, V)
    wt = table_w.reshape(F, V)
    part = _fm_scan(xt, v4, wt)
    score = _fm_combine(part)
    return score + bias[0]
